# baseline (speedup 1.0000x reference)
import functools

import jax
import jax.numpy as jnp
from jax import lax
from jax.experimental import pallas as pl
from jax.experimental.pallas import tpu as pltpu
from jax.experimental.pallas import tpu_sc as plsc

B = 16384
D = 16
H = 16
NC, NS = 2, 16
NW = NC * NS
BPW = B // NW

_sc_mesh = plsc.VectorSubcoreMesh(
    core_axis_name="c", subcore_axis_name="s", num_cores=NC, num_subcores=NS)


@functools.partial(
    pl.kernel,
    out_type=[jax.ShapeDtypeStruct((B, D), jnp.float32)] * 4,
    mesh=_sc_mesh,
    scratch_types=[
        pltpu.VMEM((BPW,), jnp.int32),
        pltpu.VMEM((BPW,), jnp.int32),
        pltpu.VMEM((BPW, D), jnp.float32),
        pltpu.VMEM((BPW, D), jnp.float32),
        pltpu.VMEM((BPW, D), jnp.float32),
        pltpu.VMEM((BPW, D), jnp.float32),
        pltpu.SemaphoreType.DMA,
        pltpu.SemaphoreType.DMA,
        pltpu.SemaphoreType.DMA,
        pltpu.SemaphoreType.DMA,
    ],
    compiler_params=pltpu.CompilerParams(use_tc_tiling_on_sc=False),
)
def _sc_gather(ug_t, ig_t, um_t, im_t, u_h, i_h, o_ug, o_ig, o_um, o_im,
               uidx, iidx, r0, r1, r2, r3, s0, s1, s2, s3):
    wid = lax.axis_index("s") * NC + lax.axis_index("c")
    base = wid * BPW
    pltpu.sync_copy(u_h.at[pl.ds(base, BPW)], uidx)
    pltpu.sync_copy(i_h.at[pl.ds(base, BPW)], iidx)
    c0 = pltpu.async_copy(ug_t.at[uidx], r0, s0)
    c1 = pltpu.async_copy(ig_t.at[iidx], r1, s1)
    c2 = pltpu.async_copy(um_t.at[uidx], r2, s2)
    c3 = pltpu.async_copy(im_t.at[iidx], r3, s3)
    c0.wait()
    c1.wait()
    c2.wait()
    c3.wait()
    pltpu.sync_copy(r0, o_ug.at[pl.ds(base, BPW)])
    pltpu.sync_copy(r1, o_ig.at[pl.ds(base, BPW)])
    pltpu.sync_copy(r2, o_um.at[pl.ds(base, BPW)])
    pltpu.sync_copy(r3, o_im.at[pl.ds(base, BPW)])


BLK = 2048


def _tc_body(ug, ig, um, im, w1t, b1, wot, bo, out):
    gmf = ug[...] * ig[...]
    mlp_in = jnp.concatenate([um[...], im[...]], axis=1)
    h = jnp.maximum(
        jnp.dot(mlp_in, w1t[...], preferred_element_type=jnp.float32) + b1[...],
        0.0)
    x = jnp.concatenate([gmf, h], axis=1)
    logit = jnp.dot(x, wot[...], preferred_element_type=jnp.float32) + bo[...]
    out[...] = 1.0 / (1.0 + jnp.exp(-logit))


_tc_mlp = pl.pallas_call(
    _tc_body,
    grid=(B // BLK,),
    in_specs=[
        pl.BlockSpec((BLK, D), lambda b: (b, 0)),
        pl.BlockSpec((BLK, D), lambda b: (b, 0)),
        pl.BlockSpec((BLK, D), lambda b: (b, 0)),
        pl.BlockSpec((BLK, D), lambda b: (b, 0)),
        pl.BlockSpec((2 * D, H), lambda b: (0, 0)),
        pl.BlockSpec((1, H), lambda b: (0, 0)),
        pl.BlockSpec((D + H, 1), lambda b: (0, 0)),
        pl.BlockSpec((1, 1), lambda b: (0, 0)),
    ],
    out_specs=pl.BlockSpec((BLK, 1), lambda b: (b, 0)),
    out_shape=jax.ShapeDtypeStruct((B, 1), jnp.float32),
)


def kernel(user_gmf, item_gmf, user_mlp, item_mlp, W1, b1, Wo, bo, u, i):
    ug, ig, um, im = _sc_gather(user_gmf, item_gmf, user_mlp, item_mlp, u, i)
    out = _tc_mlp(ug, ig, um, im, W1.T, b1.reshape(1, H), Wo.T,
                  bo.reshape(1, 1))
    return out.reshape(B)


# speedup vs baseline: 1.0014x; 1.0014x over previous
import functools

import jax
import jax.numpy as jnp
from jax import lax
from jax.experimental import pallas as pl
from jax.experimental.pallas import tpu as pltpu
from jax.experimental.pallas import tpu_sc as plsc

B = 16384
D = 16
H = 16
NROW = 1000000
NC, NS = 2, 16
NW = NC * NS
BPW = B // NW
G = BPW // D

_sc_mesh = plsc.VectorSubcoreMesh(
    core_axis_name="c", subcore_axis_name="s", num_cores=NC, num_subcores=NS)


@functools.partial(
    pl.kernel,
    out_type=[jax.ShapeDtypeStruct((D, B), jnp.float32)] * 4,
    mesh=_sc_mesh,
    scratch_types=[
        pltpu.VMEM((BPW,), jnp.int32),
        pltpu.VMEM((BPW,), jnp.int32),
        pltpu.VMEM((BPW,), jnp.int32),
        pltpu.VMEM((BPW,), jnp.int32),
        pltpu.VMEM((BPW, 8, D), jnp.float32),
        pltpu.VMEM((4, D, BPW), jnp.float32),
        pltpu.SemaphoreType.DMA,
    ],
    compiler_params=pltpu.CompilerParams(
        use_tc_tiling_on_sc=False, needs_layout_passes=False),
)
def _sc_gather(ug3, ig3, um3, im3, qu_h, qi_h, ru_h, ri_h,
               o0, o1, o2, o3, qu, qi, ru, ri, blk, sel, sem):
    wid = lax.axis_index("s") * NC + lax.axis_index("c")
    base = wid * BPW
    pltpu.sync_copy(qu_h.at[pl.ds(base, BPW)], qu)
    pltpu.sync_copy(qi_h.at[pl.ds(base, BPW)], qi)
    pltpu.sync_copy(ru_h.at[pl.ds(base, BPW)], ru)
    pltpu.sync_copy(ri_h.at[pl.ds(base, BPW)], ri)
    lane = lax.iota(jnp.int32, 16)

    for t, (tbl, q, r) in enumerate((
            (ug3, qu, ru), (ig3, qi, ri), (um3, qu, ru), (im3, qi, ri))):
        pltpu.async_copy(tbl.at[q], blk, sem).wait()

        def body(g, _, r=r, t=t):
            j16 = g * 16 + lane
            r16 = r[pl.ds(g * 16, 16)]
            for d in range(D):
                d16 = jnp.full((16,), d, jnp.int32)
                v = plsc.load_gather(blk, [j16, r16, d16])
                sel[t, d, pl.ds(g * 16, 16)] = v
            return 0

        lax.fori_loop(0, G, body, 0)

    for t, o in enumerate((o0, o1, o2, o3)):
        pltpu.sync_copy(sel.at[t], o.at[:, pl.ds(base, BPW)])


BLK = 4096


def _tc_body(ug, ig, um, im, w1, b1, wo, bo, out):
    gmf = ug[...] * ig[...]
    x = jnp.concatenate([um[...], im[...]], axis=0)
    h = jnp.maximum(
        jnp.dot(w1[...], x, preferred_element_type=jnp.float32) + b1[...],
        0.0)
    xc = jnp.concatenate([gmf, h], axis=0)
    logit = jnp.dot(wo[...], xc, preferred_element_type=jnp.float32) + bo[...]
    out[...] = 1.0 / (1.0 + jnp.exp(-logit))


_tc_mlp = pl.pallas_call(
    _tc_body,
    grid=(B // BLK,),
    in_specs=[
        pl.BlockSpec((D, BLK), lambda b: (0, b)),
        pl.BlockSpec((D, BLK), lambda b: (0, b)),
        pl.BlockSpec((D, BLK), lambda b: (0, b)),
        pl.BlockSpec((D, BLK), lambda b: (0, b)),
        pl.BlockSpec((H, 2 * D), lambda b: (0, 0)),
        pl.BlockSpec((H, 1), lambda b: (0, 0)),
        pl.BlockSpec((1, D + H), lambda b: (0, 0)),
        pl.BlockSpec((1, 1), lambda b: (0, 0)),
    ],
    out_specs=pl.BlockSpec((1, BLK), lambda b: (0, b)),
    out_shape=jax.ShapeDtypeStruct((1, B), jnp.float32),
)


def kernel(user_gmf, item_gmf, user_mlp, item_mlp, W1, b1, Wo, bo, u, i):
    ug3 = user_gmf.reshape(NROW // 8, 8, D)
    ig3 = item_gmf.reshape(NROW // 8, 8, D)
    um3 = user_mlp.reshape(NROW // 8, 8, D)
    im3 = item_mlp.reshape(NROW // 8, 8, D)
    qu = jnp.right_shift(u, 3)
    qi = jnp.right_shift(i, 3)
    ru = jnp.bitwise_and(u, 7)
    ri = jnp.bitwise_and(i, 7)
    ug, ig, um, im = _sc_gather(ug3, ig3, um3, im3, qu, qi, ru, ri)
    out = _tc_mlp(ug, ig, um, im, W1, b1.reshape(H, 1), Wo,
                  bo.reshape(1, 1))
    return out.reshape(B)
